# baseline (speedup 1.0000x reference)
import functools

import jax
import jax.numpy as jnp
from jax import lax
from jax.experimental import pallas as pl
from jax.experimental.pallas import tpu as pltpu
from jax.experimental.pallas import tpu_sc as plsc

N_FEATURES = 1024
BATCH = 16384
NC = 2
NS = 16
L = 16
NW = NC * NS
ROWS_PER_W = BATCH // NW
R = 8
NBLK = ROWS_PER_W // R
NG = N_FEATURES // L
NBUF = 4


def _permute_kernel(x_hbm, idx_hbm, out_hbm, idx_v,
                    x0, x1, x2, x3, o0, o1, o2, o3,
                    si0, si1, si2, si3, so0, so1, so2, so3):
    wid = lax.axis_index("s") * NC + lax.axis_index("c")
    row_base = wid * ROWS_PER_W
    pltpu.sync_copy(idx_hbm, idx_v)

    xb = (x0, x1, x2, x3)
    ob = (o0, o1, o2, o3)
    sib = (si0, si1, si2, si3)
    sob = (so0, so1, so2, so3)

    def in_slice(b):
        return x_hbm.at[pl.ds(row_base + b * R, R)]

    def out_slice(b):
        return out_hbm.at[pl.ds(row_base + b * R, R)]

    for p in range(NBUF):
        pltpu.async_copy(in_slice(p), xb[p], sib[p])

    def gather_block(xt, ot):
        def gather_group(g):
            goff = pl.multiple_of(g * L, L)
            col = idx_v[pl.ds(goff, L)]
            vals = []
            for r in range(R):
                rows = jnp.full((L,), r, dtype=jnp.int32)
                vals.append(plsc.load_gather(xt, [rows, col]))
            return tuple(vals)

        def store_group(g, vals):
            goff = pl.multiple_of(g * L, L)
            for r in range(R):
                ot[r, pl.ds(goff, L)] = vals[r]

        @plsc.parallel_loop(1, NG, 1, unroll=2, carry=gather_group(0))
        def g_body(g, prev):
            cur = gather_group(g)
            store_group(g - 1, prev)
            return cur

        store_group(NG - 1, g_body)

    def blk_body(b4, carry):
        for p in range(NBUF):
            xt, ot, si, so = xb[p], ob[p], sib[p], sob[p]
            b = b4 * NBUF + p
            pltpu.make_async_copy(in_slice(b), xt, si).wait()
            @pl.when(b4 > 0)
            def _drain():
                pltpu.make_async_copy(ot, out_slice(b - NBUF), so).wait()
            gather_block(xt, ot)
            pltpu.async_copy(ot, out_slice(b), so)
            @pl.when(b + NBUF < NBLK)
            def _refill():
                pltpu.async_copy(in_slice(b + NBUF), xt, si)
        return carry

    lax.fori_loop(0, NBLK // NBUF, blk_body, 0)

    for p in range(NBUF):
        pltpu.make_async_copy(ob[p], out_slice(NBLK - NBUF + p), sob[p]).wait()


@jax.jit
def kernel(x, indices):
    mesh = plsc.VectorSubcoreMesh(core_axis_name="c", subcore_axis_name="s")
    run = functools.partial(
        pl.kernel,
        mesh=mesh,
        compiler_params=pltpu.CompilerParams(needs_layout_passes=False),
        out_type=jax.ShapeDtypeStruct((BATCH, N_FEATURES), jnp.float32),
        scratch_types=(
            [pltpu.VMEM((N_FEATURES,), jnp.int32)]
            + [pltpu.VMEM((R, N_FEATURES), jnp.float32)] * (2 * NBUF)
            + [pltpu.SemaphoreType.DMA] * (2 * NBUF)
        ),
    )(_permute_kernel)
    return run(x, indices)


# speedup vs baseline: 1.0103x; 1.0103x over previous
import functools

import jax
import jax.numpy as jnp
from jax import lax
from jax.experimental import pallas as pl
from jax.experimental.pallas import tpu as pltpu
from jax.experimental.pallas import tpu_sc as plsc

N_FEATURES = 1024
BATCH = 16384
NC = 2
NS = 16
L = 16
NW = NC * NS
ROWS_PER_W = BATCH // NW
R = 8
NBLK = ROWS_PER_W // R
NG = N_FEATURES // L
NBUF = 4


def _permute_kernel(x_hbm, idx_hbm, out_hbm, idx_v,
                    x0, x1, x2, x3, o0, o1, o2, o3,
                    si0, si1, si2, si3, so0, so1, so2, so3):
    wid = lax.axis_index("s") * NC + lax.axis_index("c")
    row_base = wid * ROWS_PER_W
    pltpu.sync_copy(idx_hbm, idx_v)

    xb = (x0, x1, x2, x3)
    ob = (o0, o1, o2, o3)
    sib = (si0, si1, si2, si3)
    sob = (so0, so1, so2, so3)

    def in_slice(b):
        return x_hbm.at[pl.ds(row_base + b * R, R)]

    def out_slice(b):
        return out_hbm.at[pl.ds(row_base + b * R, R)]

    for p in range(NBUF):
        pltpu.async_copy(in_slice(p), xb[p], sib[p])

    def gather_block(xt, ot):
        def gather_group(g):
            goff = pl.multiple_of(g * L, L)
            col = idx_v[pl.ds(goff, L)]
            vals = []
            for r in range(R):
                rows = jnp.full((L,), r, dtype=jnp.int32)
                vals.append(plsc.load_gather(xt, [rows, col]))
            return tuple(vals)

        def store_group(g, vals):
            goff = pl.multiple_of(g * L, L)
            for r in range(R):
                ot[r, pl.ds(goff, L)] = vals[r]

        @plsc.parallel_loop(1, NG, 1, unroll=1, carry=gather_group(0))
        def g_body(g, prev):
            cur = gather_group(g)
            store_group(g - 1, prev)
            return cur

        store_group(NG - 1, g_body)

    def blk_body(b4, carry):
        for p in range(NBUF):
            xt, ot, si, so = xb[p], ob[p], sib[p], sob[p]
            b = b4 * NBUF + p
            pltpu.make_async_copy(in_slice(b), xt, si).wait()
            @pl.when(b4 > 0)
            def _drain():
                pltpu.make_async_copy(ot, out_slice(b - NBUF), so).wait()
            gather_block(xt, ot)
            pltpu.async_copy(ot, out_slice(b), so)
            @pl.when(b + NBUF < NBLK)
            def _refill():
                pltpu.async_copy(in_slice(b + NBUF), xt, si)
        return carry

    lax.fori_loop(0, NBLK // NBUF, blk_body, 0)

    for p in range(NBUF):
        pltpu.make_async_copy(ob[p], out_slice(NBLK - NBUF + p), sob[p]).wait()


@jax.jit
def kernel(x, indices):
    mesh = plsc.VectorSubcoreMesh(core_axis_name="c", subcore_axis_name="s")
    run = functools.partial(
        pl.kernel,
        mesh=mesh,
        compiler_params=pltpu.CompilerParams(needs_layout_passes=False),
        out_type=jax.ShapeDtypeStruct((BATCH, N_FEATURES), jnp.float32),
        scratch_types=(
            [pltpu.VMEM((N_FEATURES,), jnp.int32)]
            + [pltpu.VMEM((R, N_FEATURES), jnp.float32)] * (2 * NBUF)
            + [pltpu.SemaphoreType.DMA] * (2 * NBUF)
        ),
    )(_permute_kernel)
    return run(x, indices)
